# baseline (speedup 1.0000x reference)
import jax
import jax.numpy as jnp
from jax import lax
from jax.experimental import pallas as pl
from jax.experimental.pallas import tpu as pltpu
from jax.experimental.pallas import tpu_sc as plsc

EPSILON = 0.1
GAMMA = 0.1

NC = 2
NS = 16
NW = NC * NS
LANES = 16
EB = 32


def _tc1_body(x_ref, wencT_ref, benc_ref, wqT_ref, bq_ref, wkT_ref, bk_ref,
              wvT_ref, bv_ref, we_ref, awT_ref, basc_ref,
              k_ref, v_ref, q_ref, z0_ref, enc_ref):
    xb = x_ref[...]
    enc = jnp.dot(xb, wencT_ref[...], preferred_element_type=jnp.float32) + benc_ref[...]
    q = jnp.dot(enc, wqT_ref[...], preferred_element_type=jnp.float32) + bq_ref[...]
    k_ref[...] = jnp.dot(enc, wkT_ref[...], preferred_element_type=jnp.float32) + bk_ref[...]
    v_ref[...] = jnp.dot(enc, wvT_ref[...], preferred_element_type=jnp.float32) + bv_ref[...]
    inv_s = 1.0 / jnp.sqrt(jnp.float32(q.shape[1]))
    qw = jnp.dot(q, we_ref[...], preferred_element_type=jnp.float32)
    q_ref[...] = jnp.concatenate(
        [q * inv_s, qw * inv_s,
         jnp.zeros((q.shape[0], q.shape[1] - qw.shape[1]), jnp.float32)], axis=1)
    z0_ref[...] = jnp.dot(enc, awT_ref[...], preferred_element_type=jnp.float32) + basc_ref[...]
    enc_ref[...] = enc


def _sc1_body(lu_hbm, src_hbm, t_hbm, out_hbm, src_v, t_v, lu_b, rel_v):
    wid = lax.axis_index("s") * NC + lax.axis_index("c")
    pltpu.sync_copy(src_hbm.at[wid], src_v)
    pltpu.sync_copy(t_hbm.at[wid], t_v)
    nblk = src_v.shape[0]

    def body(blk, _):
        pltpu.sync_copy(lu_hbm.at[src_v.at[blk]], lu_b)
        for j in range(EB // LANES):
            o = pl.ds(j * LANES, LANES)
            rel_v[blk, o] = jnp.abs(lu_b[o] - t_v[blk, o])
        return 0

    lax.fori_loop(0, nblk, body, 0)
    pltpu.sync_copy(rel_v, out_hbm.at[wid])


def _tc2_body(rel_ref, msg_ref, wt_ref, bt_ref, ea_ref):
    rel = rel_ref[0, 0]
    tf = jnp.cos(rel[:, None] * wt_ref[...] + bt_ref[...])
    ea_ref[0] = jnp.concatenate([msg_ref[0], tf], axis=1)


def _sc2a_body(k_hbm, q_hbm, ea_hbm, idx_hbm, acc2_hbm, wgt_hbm,
               idx_v, k_v, q_v, ea_v, pacc_v, wrep_v, scatb_v, acc2_sh):
    cid = lax.axis_index("c")
    sid = lax.axis_index("s")
    wid = sid * NC + cid
    na = acc2_sh.shape[0]
    rpt = na // NS
    r0 = sid * rpt

    pacc_v[pl.ds(0, LANES)] = jnp.zeros((LANES,), jnp.float32)
    zv = pacc_v[pl.ds(0, LANES)]

    def zrow(e, _):
        for j in range(8):
            scatb_v[e, pl.ds(16 * j, LANES)] = zv
        return 0

    lax.fori_loop(0, EB, zrow, 0, unroll=4)

    def zcopy(i, _):
        pltpu.sync_copy(scatb_v, acc2_sh.at[pl.ds(r0 + i * EB, EB)])
        return 0

    lax.fori_loop(0, rpt // EB, zcopy, 0)
    plsc.subcore_barrier()

    nblk = idx_hbm.shape[1]

    def blk_body(blk, _):
        pltpu.sync_copy(idx_hbm.at[wid, blk], idx_v)
        pltpu.sync_copy(k_hbm.at[idx_v.at[0]], k_v)
        pltpu.sync_copy(q_hbm.at[idx_v.at[1]], q_v)
        pltpu.sync_copy(ea_hbm.at[wid, blk], ea_v)

        def edge_body(e, _):
            acc = q_v[e, pl.ds(0, LANES)] * k_v[e, pl.ds(0, LANES)]
            for j in range(1, 8):
                acc = acc + q_v[e, pl.ds(16 * j, LANES)] * k_v[e, pl.ds(16 * j, LANES)]
            for j in range(2):
                acc = acc + q_v[e, pl.ds(128 + 16 * j, LANES)] * ea_v[e, pl.ds(16 * j, LANES)]
            base = (e % 2) * 64
            pacc_v[pl.ds(base, LANES)] = acc
            for off in (8, 4, 2):
                pacc_v[pl.ds(base, LANES)] = (pacc_v[pl.ds(base, LANES)]
                                              + pacc_v[pl.ds(base + off, LANES)])
            v = pacc_v[pl.ds(base, LANES)] + pacc_v[pl.ds(base + 1, LANES)]
            pacc_v[pl.ds(base + 32, LANES)] = jnp.exp(jnp.full((LANES,), v[0], jnp.float32))
            w16 = pacc_v[pl.ds(base + 32, LANES)]
            wrep_v[e, pl.ds(0, LANES)] = w16
            for j in range(2):
                scatb_v[e, pl.ds(16 * j, LANES)] = w16 * ea_v[e, pl.ds(16 * j, LANES)]
            scatb_v[e, pl.ds(32, LANES)] = w16
            return 0

        lax.fori_loop(0, EB, edge_body, 0, unroll=2)
        pltpu.sync_copy(scatb_v, acc2_sh.at[idx_v.at[1]], add=True)
        pltpu.sync_copy(wrep_v, wgt_hbm.at[wid, blk])
        return 0

    lax.fori_loop(0, nblk, blk_body, 0)
    plsc.subcore_barrier()
    pltpu.sync_copy(acc2_sh.at[pl.ds(r0, rpt)], acc2_hbm.at[cid, pl.ds(r0, rpt)])


EB_B = 64


def _sc2b_body(v_hbm, wgt_hbm, idx_hbm, acc1_hbm,
               idx_v, v_v, wrep_v, scata_v, zb_v, acc1_sh):
    cid = lax.axis_index("c")
    sid = lax.axis_index("s")
    wid = sid * NC + cid
    na = acc1_sh.shape[0]
    rpt = na // NS
    r0 = sid * rpt

    zb_v[pl.ds(0, LANES)] = jnp.zeros((LANES,), jnp.float32)
    zv = zb_v[pl.ds(0, LANES)]

    def zrow(e, _):
        for j in range(8):
            scata_v[e, pl.ds(16 * j, LANES)] = zv
        return 0

    lax.fori_loop(0, EB_B, zrow, 0)

    def zcopy(i, _):
        pltpu.sync_copy(scata_v, acc1_sh.at[pl.ds(r0 + i * EB_B, EB_B)])
        return 0

    lax.fori_loop(0, rpt // EB_B, zcopy, 0)
    plsc.subcore_barrier()

    nblk = idx_hbm.shape[1]

    def blk_body(blk, _):
        pltpu.sync_copy(idx_hbm.at[wid, blk], idx_v)
        pltpu.sync_copy(v_hbm.at[idx_v.at[0]], v_v)
        pltpu.sync_copy(wgt_hbm.at[wid, blk], wrep_v)

        def edge_body(e, _):
            w16 = wrep_v[e, pl.ds(0, LANES)]
            for j in range(8):
                scata_v[e, pl.ds(16 * j, LANES)] = w16 * v_v[e, pl.ds(16 * j, LANES)]
            return 0

        lax.fori_loop(0, EB_B, edge_body, 0, unroll=2)
        pltpu.sync_copy(scata_v, acc1_sh.at[idx_v.at[1]], add=True)
        return 0

    lax.fori_loop(0, nblk, blk_body, 0)
    plsc.subcore_barrier()
    pltpu.sync_copy(acc1_sh.at[pl.ds(r0, rpt)], acc1_hbm.at[cid, pl.ds(r0, rpt)])


def _tc3_body(acc1_ref, acc2_ref, enc_ref, z0_ref, weT_ref, h_ref):
    num = acc1_ref[0] + acc1_ref[1]
    b = acc2_ref[0] + acc2_ref[1]
    s32 = b[:, 0:32]
    den = b[:, 32:33]
    phi = (num + jnp.dot(s32, weT_ref[...], preferred_element_type=jnp.float32)) / (den + 1e-16)
    h_ref[...] = enc_ref[...] + EPSILON * jnp.tanh(z0_ref[...] + phi)


def kernel(x, last_update, edge_index, t, msg, W_enc, b_enc, W_time, b_time,
           Wq, bq, Wk, bk, Wv, bv, We, W_asc, b_asc):
    N = x.shape[0]
    E = edge_index.shape[1]
    D = Wq.shape[0]
    MD = msg.shape[1]

    ROWB = 1280
    NP = -(-(N + NS) // ROWB) * ROWB
    EPW = -(-E // (NW * EB_B)) * EB_B
    E_pad = EPW * NW
    NBLK = EPW // EB

    f32 = jnp.float32
    aWT = W_asc.T - W_asc - GAMMA * jnp.eye(D, dtype=f32)
    x_pad = jnp.pad(x, ((0, NP - N), (0, 0)))
    pad_e = E_pad - E
    src_p = jnp.pad(edge_index[0], (0, pad_e))
    dst_p = jnp.concatenate([edge_index[1],
                             N + (jnp.arange(pad_e, dtype=jnp.int32) % NS)])
    t_p = jnp.pad(t, (0, pad_e))
    msg_p = jnp.pad(msg, ((0, pad_e), (0, 0)))

    dst4 = dst_p.reshape(NW, NBLK, EB)
    src4 = src_p.reshape(NW, NBLK, EB)
    idx4 = jnp.stack([src4, dst4], axis=2)
    t3 = t_p.reshape(NW, NBLK, EB)
    msg3 = msg_p.reshape(NW, EPW, MD)

    full = lambda s: pl.BlockSpec(s, lambda i: (0,) * len(s))
    rowb = lambda w: pl.BlockSpec((ROWB, w), lambda i: (i, 0))
    kt, vt, qt, z0, enc = pl.pallas_call(
        _tc1_body,
        grid=(NP // ROWB,),
        in_specs=[rowb(2 * D), full((2 * D, D)), full((1, D)), full((D, D)),
                  full((1, D)), full((D, D)), full((1, D)), full((D, D)),
                  full((1, D)), full((D, 2 * MD)), full((D, D)), full((1, D))],
        out_specs=[rowb(D), rowb(D), rowb(2 * D), rowb(D), rowb(D)],
        out_shape=[jax.ShapeDtypeStruct((NP, D), f32),
                   jax.ShapeDtypeStruct((NP, D), f32),
                   jax.ShapeDtypeStruct((NP, 2 * D), f32),
                   jax.ShapeDtypeStruct((NP, D), f32),
                   jax.ShapeDtypeStruct((NP, D), f32)],
    )(x_pad, W_enc.T, b_enc[None, :], Wq.T, bq[None, :], Wk.T, bk[None, :],
      Wv.T, bv[None, :], We, aWT, b_asc[None, :])

    sc_mesh = plsc.VectorSubcoreMesh(core_axis_name="c", subcore_axis_name="s",
                                     num_cores=NC, num_subcores=NS)
    rel2 = pl.kernel(
        _sc1_body,
        out_type=jax.ShapeDtypeStruct((NW, NBLK, EB), f32),
        mesh=sc_mesh,
        scratch_types=[pltpu.VMEM((NBLK, EB), jnp.int32),
                       pltpu.VMEM((NBLK, EB), f32),
                       pltpu.VMEM((EB,), f32),
                       pltpu.VMEM((NBLK, EB), f32)],
    )(last_update, src4, t3)

    ea3 = pl.pallas_call(
        _tc2_body,
        grid=(NW,),
        in_specs=[pl.BlockSpec((1, 1, EPW), lambda i: (i, 0, 0)),
                  pl.BlockSpec((1, EPW, MD), lambda i: (i, 0, 0)),
                  full((1, MD)), full((1, MD))],
        out_specs=pl.BlockSpec((1, EPW, 2 * MD), lambda i: (i, 0, 0)),
        out_shape=jax.ShapeDtypeStruct((NW, EPW, 2 * MD), f32),
    )(rel2.reshape(NW, 1, EPW), msg3, W_time[:, 0][None, :], b_time[None, :])
    ea4 = ea3.reshape(NW, NBLK, EB, 2 * MD)

    acc2, wgt = pl.kernel(
        _sc2a_body,
        out_type=[jax.ShapeDtypeStruct((NC, NP, D), f32),
                  jax.ShapeDtypeStruct((NW, NBLK, EB, LANES), f32)],
        mesh=sc_mesh,
        scratch_types=[
            pltpu.VMEM((2, EB), jnp.int32),
            pltpu.VMEM((EB, D), f32),
            pltpu.VMEM((EB, 2 * D), f32),
            pltpu.VMEM((EB, 2 * MD), f32),
            pltpu.VMEM((128,), f32),
            pltpu.VMEM((EB, LANES), f32),
            pltpu.VMEM((EB, D), f32),
            pltpu.VMEM_SHARED((NP, D), f32),
        ],
    )(kt, qt, ea4, idx4)

    NBLK_B = EPW // EB_B
    idxb = jnp.stack([src_p.reshape(NW, NBLK_B, EB_B),
                      dst_p.reshape(NW, NBLK_B, EB_B)], axis=2)
    wgt_b = wgt.reshape(NW, NBLK_B, EB_B, LANES)
    acc1 = pl.kernel(
        _sc2b_body,
        out_type=jax.ShapeDtypeStruct((NC, NP, D), f32),
        mesh=sc_mesh,
        scratch_types=[
            pltpu.VMEM((2, EB_B), jnp.int32),
            pltpu.VMEM((EB_B, D), f32),
            pltpu.VMEM((EB_B, LANES), f32),
            pltpu.VMEM((EB_B, D), f32),
            pltpu.VMEM((LANES,), f32),
            pltpu.VMEM_SHARED((NP, D), f32),
        ],
    )(vt, wgt_b, idxb)

    h = pl.pallas_call(
        _tc3_body,
        grid=(1,),
        in_specs=[full((NC, NP, D)), full((NC, NP, D)), full((NP, D)),
                  full((NP, D)), full((2 * MD, D))],
        out_specs=full((NP, D)),
        out_shape=jax.ShapeDtypeStruct((NP, D), f32),
    )(acc1, acc2, enc, z0, We.T)
    return h[:N]


# speedup vs baseline: 1.0953x; 1.0953x over previous
import jax
import jax.numpy as jnp
from jax import lax
from jax.experimental import pallas as pl
from jax.experimental.pallas import tpu as pltpu
from jax.experimental.pallas import tpu_sc as plsc

EPSILON = 0.1
GAMMA = 0.1

NC = 2
NS = 16
NW = NC * NS
LANES = 16
EB = 32


def _tc1_body(x_ref, wencT_ref, benc_ref, wqT_ref, bq_ref, wkT_ref, bk_ref,
              wvT_ref, bv_ref, we_ref, awT_ref, basc_ref,
              k_ref, v_ref, q_ref, z0_ref, enc_ref):
    xb = x_ref[...]
    enc = jnp.dot(xb, wencT_ref[...], preferred_element_type=jnp.float32) + benc_ref[...]
    q = jnp.dot(enc, wqT_ref[...], preferred_element_type=jnp.float32) + bq_ref[...]
    k_ref[...] = jnp.dot(enc, wkT_ref[...], preferred_element_type=jnp.float32) + bk_ref[...]
    v_ref[...] = jnp.dot(enc, wvT_ref[...], preferred_element_type=jnp.float32) + bv_ref[...]
    inv_s = 1.0 / jnp.sqrt(jnp.float32(q.shape[1]))
    qw = jnp.dot(q, we_ref[...], preferred_element_type=jnp.float32)
    q_ref[...] = jnp.concatenate(
        [q * inv_s, qw * inv_s,
         jnp.zeros((q.shape[0], q.shape[1] - qw.shape[1]), jnp.float32)], axis=1)
    z0_ref[...] = jnp.dot(enc, awT_ref[...], preferred_element_type=jnp.float32) + basc_ref[...]
    enc_ref[...] = enc


def _sc1_body(lu_hbm, src_hbm, t_hbm, out_hbm, src_v, t_v, lu_b, rel_v):
    wid = lax.axis_index("s") * NC + lax.axis_index("c")
    pltpu.sync_copy(src_hbm.at[wid], src_v)
    pltpu.sync_copy(t_hbm.at[wid], t_v)
    nblk = src_v.shape[0]

    def body(blk, _):
        pltpu.sync_copy(lu_hbm.at[src_v.at[blk]], lu_b)
        for j in range(EB // LANES):
            o = pl.ds(j * LANES, LANES)
            rel_v[blk, o] = jnp.abs(lu_b[o] - t_v[blk, o])
        return 0

    lax.fori_loop(0, nblk, body, 0)
    pltpu.sync_copy(rel_v, out_hbm.at[wid])


def _tc2_body(rel_ref, msg_ref, wt_ref, bt_ref, ea_ref):
    rel = rel_ref[0, 0]
    tf = jnp.cos(rel[:, None] * wt_ref[...] + bt_ref[...])
    ea_ref[0] = jnp.concatenate([msg_ref[0], tf], axis=1)


def _sc2a_body(k_hbm, q_hbm, ea_hbm, idx_hbm, acc2_hbm, wgt_hbm,
               idx_v, k_v, q_v, ea_v, pacc_v, wrep_v, scatb_v, acc2_sh):
    cid = lax.axis_index("c")
    sid = lax.axis_index("s")
    wid = sid * NC + cid
    na = acc2_sh.shape[0]
    rpt = na // NS
    r0 = sid * rpt

    pacc_v[pl.ds(0, LANES)] = jnp.zeros((LANES,), jnp.float32)
    zv = pacc_v[pl.ds(0, LANES)]

    def zrow(e, _):
        for j in range(8):
            scatb_v[e, pl.ds(16 * j, LANES)] = zv
        return 0

    lax.fori_loop(0, EB, zrow, 0, unroll=4)

    def zcopy(i, _):
        pltpu.sync_copy(scatb_v, acc2_sh.at[pl.ds(r0 + i * EB, EB)])
        return 0

    lax.fori_loop(0, rpt // EB, zcopy, 0)
    plsc.subcore_barrier()

    nblk = idx_hbm.shape[1]

    def blk_body(blk, _):
        pltpu.sync_copy(idx_hbm.at[wid, blk], idx_v)
        pltpu.sync_copy(k_hbm.at[idx_v.at[0]], k_v)
        pltpu.sync_copy(q_hbm.at[idx_v.at[1]], q_v)
        pltpu.sync_copy(ea_hbm.at[wid, blk], ea_v)

        def edge_body(e, _):
            acc = q_v[e, pl.ds(0, LANES)] * k_v[e, pl.ds(0, LANES)]
            for j in range(1, 8):
                acc = acc + q_v[e, pl.ds(16 * j, LANES)] * k_v[e, pl.ds(16 * j, LANES)]
            for j in range(2):
                acc = acc + q_v[e, pl.ds(128 + 16 * j, LANES)] * ea_v[e, pl.ds(16 * j, LANES)]
            base = (e % 2) * 64
            pacc_v[pl.ds(base, LANES)] = acc
            for off in (8, 4, 2):
                pacc_v[pl.ds(base, LANES)] = (pacc_v[pl.ds(base, LANES)]
                                              + pacc_v[pl.ds(base + off, LANES)])
            v = pacc_v[pl.ds(base, LANES)] + pacc_v[pl.ds(base + 1, LANES)]
            pacc_v[pl.ds(base + 32, LANES)] = jnp.exp(jnp.full((LANES,), v[0], jnp.float32))
            w16 = pacc_v[pl.ds(base + 32, LANES)]
            wrep_v[e, pl.ds(0, LANES)] = w16
            for j in range(2):
                scatb_v[e, pl.ds(16 * j, LANES)] = w16 * ea_v[e, pl.ds(16 * j, LANES)]
            scatb_v[e, pl.ds(32, LANES)] = w16
            return 0

        lax.fori_loop(0, EB, edge_body, 0)
        pltpu.sync_copy(scatb_v, acc2_sh.at[idx_v.at[1]], add=True)
        pltpu.sync_copy(wrep_v, wgt_hbm.at[wid, blk])
        return 0

    lax.fori_loop(0, nblk, blk_body, 0)
    plsc.subcore_barrier()
    pltpu.sync_copy(acc2_sh.at[pl.ds(r0, rpt)], acc2_hbm.at[cid, pl.ds(r0, rpt)])


EB_B = 64


def _sc2b_body(v_hbm, wgt_hbm, idx_hbm, acc1_hbm,
               idx_v, v_v, wrep_v, scata_v, zb_v, acc1_sh):
    cid = lax.axis_index("c")
    sid = lax.axis_index("s")
    wid = sid * NC + cid
    na = acc1_sh.shape[0]
    rpt = na // NS
    r0 = sid * rpt

    zb_v[pl.ds(0, LANES)] = jnp.zeros((LANES,), jnp.float32)
    zv = zb_v[pl.ds(0, LANES)]

    def zrow(e, _):
        for j in range(8):
            scata_v[e, pl.ds(16 * j, LANES)] = zv
        return 0

    lax.fori_loop(0, EB_B, zrow, 0)

    def zcopy(i, _):
        pltpu.sync_copy(scata_v, acc1_sh.at[pl.ds(r0 + i * EB_B, EB_B)])
        return 0

    lax.fori_loop(0, rpt // EB_B, zcopy, 0)
    plsc.subcore_barrier()

    nblk = idx_hbm.shape[1]

    def blk_body(blk, _):
        pltpu.sync_copy(idx_hbm.at[wid, blk], idx_v)
        pltpu.sync_copy(v_hbm.at[idx_v.at[0]], v_v)
        pltpu.sync_copy(wgt_hbm.at[wid, blk], wrep_v)

        def edge_body(e, _):
            w16 = wrep_v[e, pl.ds(0, LANES)]
            for j in range(8):
                scata_v[e, pl.ds(16 * j, LANES)] = w16 * v_v[e, pl.ds(16 * j, LANES)]
            return 0

        lax.fori_loop(0, EB_B, edge_body, 0)
        pltpu.sync_copy(scata_v, acc1_sh.at[idx_v.at[1]], add=True)
        return 0

    lax.fori_loop(0, nblk, blk_body, 0)
    plsc.subcore_barrier()
    pltpu.sync_copy(acc1_sh.at[pl.ds(r0, rpt)], acc1_hbm.at[cid, pl.ds(r0, rpt)])


def _tc3_body(acc1_ref, acc2_ref, enc_ref, z0_ref, weT_ref, h_ref):
    num = acc1_ref[0] + acc1_ref[1]
    b = acc2_ref[0] + acc2_ref[1]
    s32 = b[:, 0:32]
    den = b[:, 32:33]
    phi = (num + jnp.dot(s32, weT_ref[...], preferred_element_type=jnp.float32)) / (den + 1e-16)
    h_ref[...] = enc_ref[...] + EPSILON * jnp.tanh(z0_ref[...] + phi)


def kernel(x, last_update, edge_index, t, msg, W_enc, b_enc, W_time, b_time,
           Wq, bq, Wk, bk, Wv, bv, We, W_asc, b_asc):
    N = x.shape[0]
    E = edge_index.shape[1]
    D = Wq.shape[0]
    MD = msg.shape[1]

    ROWB = 1280
    NP = -(-(N + NS) // ROWB) * ROWB
    EPW = -(-E // (NW * EB_B)) * EB_B
    E_pad = EPW * NW
    NBLK = EPW // EB

    f32 = jnp.float32
    aWT = W_asc.T - W_asc - GAMMA * jnp.eye(D, dtype=f32)
    x_pad = jnp.pad(x, ((0, NP - N), (0, 0)))
    pad_e = E_pad - E
    src_p = jnp.pad(edge_index[0], (0, pad_e))
    dst_p = jnp.concatenate([edge_index[1],
                             N + (jnp.arange(pad_e, dtype=jnp.int32) % NS)])
    t_p = jnp.pad(t, (0, pad_e))
    msg_p = jnp.pad(msg, ((0, pad_e), (0, 0)))

    dst4 = dst_p.reshape(NW, NBLK, EB)
    src4 = src_p.reshape(NW, NBLK, EB)
    idx4 = jnp.stack([src4, dst4], axis=2)
    t3 = t_p.reshape(NW, NBLK, EB)
    msg3 = msg_p.reshape(NW, EPW, MD)

    full = lambda s: pl.BlockSpec(s, lambda i: (0,) * len(s))
    rowb = lambda w: pl.BlockSpec((ROWB, w), lambda i: (i, 0))
    kt, vt, qt, z0, enc = pl.pallas_call(
        _tc1_body,
        grid=(NP // ROWB,),
        in_specs=[rowb(2 * D), full((2 * D, D)), full((1, D)), full((D, D)),
                  full((1, D)), full((D, D)), full((1, D)), full((D, D)),
                  full((1, D)), full((D, 2 * MD)), full((D, D)), full((1, D))],
        out_specs=[rowb(D), rowb(D), rowb(2 * D), rowb(D), rowb(D)],
        out_shape=[jax.ShapeDtypeStruct((NP, D), f32),
                   jax.ShapeDtypeStruct((NP, D), f32),
                   jax.ShapeDtypeStruct((NP, 2 * D), f32),
                   jax.ShapeDtypeStruct((NP, D), f32),
                   jax.ShapeDtypeStruct((NP, D), f32)],
    )(x_pad, W_enc.T, b_enc[None, :], Wq.T, bq[None, :], Wk.T, bk[None, :],
      Wv.T, bv[None, :], We, aWT, b_asc[None, :])

    sc_mesh = plsc.VectorSubcoreMesh(core_axis_name="c", subcore_axis_name="s",
                                     num_cores=NC, num_subcores=NS)
    rel2 = pl.kernel(
        _sc1_body,
        out_type=jax.ShapeDtypeStruct((NW, NBLK, EB), f32),
        mesh=sc_mesh,
        scratch_types=[pltpu.VMEM((NBLK, EB), jnp.int32),
                       pltpu.VMEM((NBLK, EB), f32),
                       pltpu.VMEM((EB,), f32),
                       pltpu.VMEM((NBLK, EB), f32)],
    )(last_update, src4, t3)

    ea3 = pl.pallas_call(
        _tc2_body,
        grid=(NW,),
        in_specs=[pl.BlockSpec((1, 1, EPW), lambda i: (i, 0, 0)),
                  pl.BlockSpec((1, EPW, MD), lambda i: (i, 0, 0)),
                  full((1, MD)), full((1, MD))],
        out_specs=pl.BlockSpec((1, EPW, 2 * MD), lambda i: (i, 0, 0)),
        out_shape=jax.ShapeDtypeStruct((NW, EPW, 2 * MD), f32),
    )(rel2.reshape(NW, 1, EPW), msg3, W_time[:, 0][None, :], b_time[None, :])
    ea4 = ea3.reshape(NW, NBLK, EB, 2 * MD)

    acc2, wgt = pl.kernel(
        _sc2a_body,
        out_type=[jax.ShapeDtypeStruct((NC, NP, D), f32),
                  jax.ShapeDtypeStruct((NW, NBLK, EB, LANES), f32)],
        mesh=sc_mesh,
        scratch_types=[
            pltpu.VMEM((2, EB), jnp.int32),
            pltpu.VMEM((EB, D), f32),
            pltpu.VMEM((EB, 2 * D), f32),
            pltpu.VMEM((EB, 2 * MD), f32),
            pltpu.VMEM((128,), f32),
            pltpu.VMEM((EB, LANES), f32),
            pltpu.VMEM((EB, D), f32),
            pltpu.VMEM_SHARED((NP, D), f32),
        ],
    )(kt, qt, ea4, idx4)

    NBLK_B = EPW // EB_B
    idxb = jnp.stack([src_p.reshape(NW, NBLK_B, EB_B),
                      dst_p.reshape(NW, NBLK_B, EB_B)], axis=2)
    wgt_b = wgt.reshape(NW, NBLK_B, EB_B, LANES)
    acc1 = pl.kernel(
        _sc2b_body,
        out_type=jax.ShapeDtypeStruct((NC, NP, D), f32),
        mesh=sc_mesh,
        scratch_types=[
            pltpu.VMEM((2, EB_B), jnp.int32),
            pltpu.VMEM((EB_B, D), f32),
            pltpu.VMEM((EB_B, LANES), f32),
            pltpu.VMEM((EB_B, D), f32),
            pltpu.VMEM((LANES,), f32),
            pltpu.VMEM_SHARED((NP, D), f32),
        ],
    )(vt, wgt_b, idxb)

    h = pl.pallas_call(
        _tc3_body,
        grid=(1,),
        in_specs=[full((NC, NP, D)), full((NC, NP, D)), full((NP, D)),
                  full((NP, D)), full((2 * MD, D))],
        out_specs=full((NP, D)),
        out_shape=jax.ShapeDtypeStruct((NP, D), f32),
    )(acc1, acc2, enc, z0, We.T)
    return h[:N]
